# baseline (speedup 1.0000x reference)
import functools

import jax
import jax.numpy as jnp
from jax import lax
from jax.experimental import pallas as pl
from jax.experimental.pallas import tpu as pltpu
from jax.experimental.pallas import tpu_sc as plsc

BATCH = 16384
HIDDEN = 64

NUM_CORES = 2
NUM_SUBCORES = 16
NUM_WORKERS = NUM_CORES * NUM_SUBCORES

B_PER_W = BATCH // NUM_WORKERS
CHUNK = 128
N_CHUNKS = B_PER_W // CHUNK


def _gather_body(table_hbm, idx_hbm, out_hbm, idx_v, rows_v, sem):
    wid = lax.axis_index("s") * NUM_CORES + lax.axis_index("c")
    pltpu.sync_copy(idx_hbm.at[pl.ds(wid * N_CHUNKS, N_CHUNKS)], idx_v)
    copies = [
        pltpu.async_copy(table_hbm.at[idx_v.at[j]], rows_v.at[j], sem)
        for j in range(N_CHUNKS)
    ]
    for c in copies:
        c.wait()
    pltpu.sync_copy(rows_v, out_hbm.at[pl.ds(wid * N_CHUNKS, N_CHUNKS)])


_gather_call = functools.partial(
    pl.kernel,
    out_type=jax.ShapeDtypeStruct((BATCH // CHUNK, CHUNK, HIDDEN), jnp.float32),
    mesh=plsc.VectorSubcoreMesh(core_axis_name="c", subcore_axis_name="s"),
    scratch_types=[
        pltpu.VMEM((N_CHUNKS, CHUNK), jnp.int32),
        pltpu.VMEM((N_CHUNKS, CHUNK, HIDDEN), jnp.float32),
        pltpu.SemaphoreType.DMA,
    ],
    compiler_params=pltpu.CompilerParams(use_tc_tiling_on_sc=False),
)(_gather_body)


@jax.jit
def kernel(labels, embedding_table):
    idx = labels.astype(jnp.int32).reshape(BATCH // CHUNK, CHUNK)
    out = _gather_call(embedding_table, idx)
    return out.reshape(BATCH, HIDDEN)


# speedup vs baseline: 1.6320x; 1.6320x over previous
import functools

import jax
import jax.numpy as jnp
from jax import lax
from jax.experimental import pallas as pl
from jax.experimental.pallas import tpu as pltpu
from jax.experimental.pallas import tpu_sc as plsc

BATCH = 16384
HIDDEN = 64

NUM_CORES = 2
NUM_SUBCORES = 16
NUM_WORKERS = NUM_CORES * NUM_SUBCORES

B_PER_W = BATCH // NUM_WORKERS
K = 16
N_BATCHES = B_PER_W // K


def _gather_body(table_hbm, idx_hbm, out_hbm, idx_v, rows_v, sem):
    wid = lax.axis_index("s") * NUM_CORES + lax.axis_index("c")
    base = wid * B_PER_W
    pltpu.sync_copy(idx_hbm.at[pl.ds(base, B_PER_W)], idx_v)

    def batch(g, carry):
        gb = g * K
        vlbl = idx_v[pl.ds(gb, K)]
        copies = []
        for j in range(K):
            lbl = vlbl[j]
            copies.append(
                pltpu.async_copy(
                    table_hbm.at[pl.ds(lbl, 1)],
                    rows_v.at[pl.ds(gb + j, 1)],
                    sem,
                )
            )
        for c in copies:
            c.wait()
        return carry

    lax.fori_loop(0, N_BATCHES, batch, 0)
    pltpu.sync_copy(rows_v, out_hbm.at[pl.ds(base, B_PER_W)])


_gather_call = functools.partial(
    pl.kernel,
    out_type=jax.ShapeDtypeStruct((BATCH, HIDDEN), jnp.float32),
    mesh=plsc.VectorSubcoreMesh(core_axis_name="c", subcore_axis_name="s"),
    scratch_types=[
        pltpu.VMEM((B_PER_W,), jnp.int32),
        pltpu.VMEM((B_PER_W, HIDDEN), jnp.float32),
        pltpu.SemaphoreType.DMA,
    ],
    compiler_params=pltpu.CompilerParams(use_tc_tiling_on_sc=True),
)(_gather_body)


@jax.jit
def kernel(labels, embedding_table):
    return _gather_call(embedding_table, labels.astype(jnp.int32))


# speedup vs baseline: 2.8049x; 1.7187x over previous
import functools

import jax
import jax.numpy as jnp
from jax import lax
from jax.experimental import pallas as pl
from jax.experimental.pallas import tpu as pltpu
from jax.experimental.pallas import tpu_sc as plsc

BATCH = 16384
HIDDEN = 64

NUM_CORES = 2
NUM_SUBCORES = 16
NUM_WORKERS = NUM_CORES * NUM_SUBCORES
LANES = 16

B_PER_W = BATCH // NUM_WORKERS
NBUF = 4
GROUPS = B_PER_W // LANES


def _gather_body(table_t_hbm, idx_hbm, out_t_hbm, idx_v, blk_v, cols_v, sem):
    wid = lax.axis_index("s") * NUM_CORES + lax.axis_index("c")
    base = wid * B_PER_W
    pltpu.sync_copy(idx_hbm.at[pl.ds(base, B_PER_W)], idx_v)

    lane_iota = lax.iota(jnp.int32, LANES)

    def fetch(slot, lbl):
        q128 = lax.shift_right_logical(lbl, 7) * 128
        return pltpu.async_copy(
            table_t_hbm.at[:, pl.ds(q128, 128)],
            blk_v.at[slot],
            sem,
        )

    def extract(slot, lbl, b):
        r = lbl & 127
        col = jnp.broadcast_to(r, (LANES,))
        dstc = jnp.broadcast_to(b, (LANES,))
        for g in range(HIDDEN // LANES):
            rows = lane_iota + (g * LANES)
            vals = plsc.load_gather(blk_v.at[slot], [rows, col])
            plsc.store_scatter(cols_v, [rows, dstc], vals)

    def group(gi, carry):
        gb = gi * LANES
        vlbl = idx_v[pl.ds(gb, LANES)]
        copies = [fetch(k, vlbl[k]) for k in range(NBUF)]
        for k in range(LANES):
            copies[k % NBUF].wait()
            extract(k % NBUF, vlbl[k], gb + k)
            if k + NBUF < LANES:
                copies[(k + NBUF) % NBUF] = fetch((k + NBUF) % NBUF,
                                                  vlbl[k + NBUF])
        return carry

    lax.fori_loop(0, GROUPS, group, 0)
    pltpu.sync_copy(cols_v, out_t_hbm.at[:, pl.ds(base, B_PER_W)])


_gather_call = functools.partial(
    pl.kernel,
    out_type=jax.ShapeDtypeStruct((HIDDEN, BATCH), jnp.float32),
    mesh=plsc.VectorSubcoreMesh(core_axis_name="c", subcore_axis_name="s"),
    scratch_types=[
        pltpu.VMEM((B_PER_W,), jnp.int32),
        pltpu.VMEM((NBUF, HIDDEN, 128), jnp.float32),
        pltpu.VMEM((HIDDEN, B_PER_W), jnp.float32),
        pltpu.SemaphoreType.DMA,
    ],
    compiler_params=pltpu.CompilerParams(
        use_tc_tiling_on_sc=True, needs_layout_passes=False
    ),
)(_gather_body)


@jax.jit
def kernel(labels, embedding_table):
    out_t = _gather_call(embedding_table.T, labels.astype(jnp.int32))
    return out_t.T


# speedup vs baseline: 3.0294x; 1.0800x over previous
import functools

import jax
import jax.numpy as jnp
from jax import lax
from jax.experimental import pallas as pl
from jax.experimental.pallas import tpu as pltpu
from jax.experimental.pallas import tpu_sc as plsc

BATCH = 16384
HIDDEN = 64

NUM_CORES = 2
NUM_SUBCORES = 16
NUM_WORKERS = NUM_CORES * NUM_SUBCORES
LANES = 16

B_PER_W = BATCH // NUM_WORKERS
NBUF = 4
GROUPS = B_PER_W // LANES


def _gather_body(table_t_hbm, idx_hbm, out_t_hbm, idx_v, blk_v, cols_v, sem):
    wid = lax.axis_index("s") * NUM_CORES + lax.axis_index("c")
    base = wid * B_PER_W
    pltpu.sync_copy(idx_hbm.at[pl.ds(base, B_PER_W)], idx_v)

    lane_iota = lax.iota(jnp.int32, LANES)

    def fetch(slot, lbl):
        q128 = lax.shift_right_logical(lbl, 7) * 128
        return pltpu.async_copy(
            table_t_hbm.at[:, pl.ds(q128, 128)],
            blk_v.at[slot],
            sem,
        )

    def drain(slot):
        pltpu.make_async_copy(
            table_t_hbm.at[:, pl.ds(0, 128)], blk_v.at[slot], sem
        ).wait()

    def extract(slot, lbl, b):
        r = lbl & 127
        col = jnp.broadcast_to(r, (LANES,))
        dstc = jnp.broadcast_to(b, (LANES,))
        for g in range(HIDDEN // LANES):
            rows = lane_iota + (g * LANES)
            vals = plsc.load_gather(blk_v.at[slot], [rows, col])
            plsc.store_scatter(cols_v, [rows, dstc], vals)

    vlbl0 = idx_v[pl.ds(0, LANES)]
    for k in range(NBUF):
        fetch(k, vlbl0[k])

    def group(gi, vlbl):
        gb = gi * LANES
        nxt = (gi + 1) * LANES
        vnxt = idx_v[pl.ds(jnp.where(nxt < B_PER_W, nxt, 0), LANES)]
        for k in range(LANES):
            slot = k % NBUF
            drain(slot)
            extract(slot, vlbl[k], gb + k)
            if k + NBUF < LANES:
                fetch(slot, vlbl[k + NBUF])
            else:
                fetch(slot, vnxt[k + NBUF - LANES])
        return vnxt

    vlast = lax.fori_loop(0, GROUPS - 1, group, vlbl0)
    gb = (GROUPS - 1) * LANES
    for k in range(LANES):
        slot = k % NBUF
        drain(slot)
        extract(slot, vlast[k], gb + k)
        if k + NBUF < LANES:
            fetch(slot, vlast[k + NBUF])
    pltpu.sync_copy(cols_v, out_t_hbm.at[:, pl.ds(base, B_PER_W)])


_gather_call = functools.partial(
    pl.kernel,
    out_type=jax.ShapeDtypeStruct((HIDDEN, BATCH), jnp.float32),
    mesh=plsc.VectorSubcoreMesh(core_axis_name="c", subcore_axis_name="s"),
    scratch_types=[
        pltpu.VMEM((B_PER_W,), jnp.int32),
        pltpu.VMEM((NBUF, HIDDEN, 128), jnp.float32),
        pltpu.VMEM((HIDDEN, B_PER_W), jnp.float32),
        pltpu.SemaphoreType.DMA,
    ],
    compiler_params=pltpu.CompilerParams(
        use_tc_tiling_on_sc=True, needs_layout_passes=False
    ),
)(_gather_body)


@jax.jit
def kernel(labels, embedding_table):
    out_t = _gather_call(embedding_table.T, labels.astype(jnp.int32))
    return out_t.T


# speedup vs baseline: 3.2104x; 1.0597x over previous
import functools

import jax
import jax.numpy as jnp
from jax import lax
from jax.experimental import pallas as pl
from jax.experimental.pallas import tpu as pltpu
from jax.experimental.pallas import tpu_sc as plsc

BATCH = 16384
HIDDEN = 64

NUM_CORES = 2
NUM_SUBCORES = 16
NUM_WORKERS = NUM_CORES * NUM_SUBCORES
LANES = 16

B_PER_W = BATCH // NUM_WORKERS
NBUF = 8
GROUPS = B_PER_W // LANES


def _gather_body(table_t_hbm, idx_hbm, out_t_hbm, idx_v, blk_v, cols_v, sem):
    wid = lax.axis_index("s") * NUM_CORES + lax.axis_index("c")
    base = wid * B_PER_W
    pltpu.sync_copy(idx_hbm.at[pl.ds(base, B_PER_W)], idx_v)

    lane_iota = lax.iota(jnp.int32, LANES)

    def fetch(slot, lbl):
        q128 = lax.shift_right_logical(lbl, 7) * 128
        return pltpu.async_copy(
            table_t_hbm.at[:, pl.ds(q128, 128)],
            blk_v.at[slot],
            sem,
        )

    def drain(slot):
        pltpu.make_async_copy(
            table_t_hbm.at[:, pl.ds(0, 128)], blk_v.at[slot], sem
        ).wait()

    def extract(slot, lbl, b):
        r = lbl & 127
        col = jnp.broadcast_to(r, (LANES,))
        dstc = jnp.broadcast_to(b, (LANES,))
        for g in range(HIDDEN // LANES):
            rows = lane_iota + (g * LANES)
            vals = plsc.load_gather(blk_v.at[slot], [rows, col])
            plsc.store_scatter(cols_v, [rows, dstc], vals)

    vlbl0 = idx_v[pl.ds(0, LANES)]
    for k in range(NBUF):
        fetch(k, vlbl0[k])

    def group(gi, vlbl):
        gb = gi * LANES
        nxt = (gi + 1) * LANES
        vnxt = idx_v[pl.ds(jnp.where(nxt < B_PER_W, nxt, 0), LANES)]
        for k in range(LANES):
            slot = k % NBUF
            drain(slot)
            extract(slot, vlbl[k], gb + k)
            if k + NBUF < LANES:
                fetch(slot, vlbl[k + NBUF])
            else:
                fetch(slot, vnxt[k + NBUF - LANES])
        return vnxt

    vlast = lax.fori_loop(0, GROUPS - 1, group, vlbl0)
    gb = (GROUPS - 1) * LANES
    for k in range(LANES):
        slot = k % NBUF
        drain(slot)
        extract(slot, vlast[k], gb + k)
        if k + NBUF < LANES:
            fetch(slot, vlast[k + NBUF])
    pltpu.sync_copy(cols_v, out_t_hbm.at[:, pl.ds(base, B_PER_W)])


_gather_call = functools.partial(
    pl.kernel,
    out_type=jax.ShapeDtypeStruct((HIDDEN, BATCH), jnp.float32),
    mesh=plsc.VectorSubcoreMesh(core_axis_name="c", subcore_axis_name="s"),
    scratch_types=[
        pltpu.VMEM((B_PER_W,), jnp.int32),
        pltpu.VMEM((NBUF, HIDDEN, 128), jnp.float32),
        pltpu.VMEM((HIDDEN, B_PER_W), jnp.float32),
        pltpu.SemaphoreType.DMA,
    ],
    compiler_params=pltpu.CompilerParams(
        use_tc_tiling_on_sc=True, needs_layout_passes=False
    ),
)(_gather_body)


@jax.jit
def kernel(labels, embedding_table):
    out_t = _gather_call(embedding_table.T, labels.astype(jnp.int32))
    return out_t.T
